# baseline (speedup 1.0000x reference)
import jax
import jax.numpy as jnp
from jax.experimental import pallas as pl
from jax.experimental.pallas import tpu as pltpu

_B, _N, _S = 8, 2048, 512
_C1, _C2 = 128, 256
_H1, _H2 = 256, 128
_BN = _B * _N
_CN = 256


def _fp_kernel(xyz1_ref, xyz2t_ref, points1_ref, points2_ref,
               w1_ref, b1_ref, g1_ref, be1_ref,
               w2_ref, b2_ref, g2_ref, be2_ref,
               out_ref, h1_ref, h2_ref, s1_ref, q1_ref, s2_ref, q2_ref,
               w1t_ref, w2t_ref, x1t_ref):
    p = pl.program_id(0)
    b = pl.program_id(1)

    @pl.when(jnp.logical_and(p == 0, b == 0))
    def _transpose_weights():
        w1t_ref[...] = jnp.transpose(w1_ref[...])
        w2t_ref[...] = jnp.transpose(w2_ref[...])

    @pl.when(p == 0)
    def _phase0():
        x1t_ref[...] = jnp.transpose(xyz1_ref[0])
        x2 = xyz2t_ref[0]
        xc0 = x2[:, 0:1]
        xc1 = x2[:, 1:2]
        xc2 = x2[:, 2:3]
        big = jnp.float32(3.0e38)
        zero = jnp.float32(0.0)

        def chunk(i):
            x1c = x1t_ref[:, pl.ds(i * _CN, _CN)]
            dx = xc0 - x1c[0:1, :]
            dy = xc1 - x1c[1:2, :]
            dz = xc2 - x1c[2:3, :]
            dsq = dx * dx + dy * dy + dz * dz

            m1s = jnp.min(dsq, axis=0, keepdims=True)
            eq1 = dsq <= m1s
            t2 = jnp.where(eq1, big, dsq)
            m2s = jnp.min(t2, axis=0, keepdims=True)
            eq2 = t2 <= m2s
            t3 = jnp.where(eq2, big, t2)
            m3s = jnp.min(t3, axis=0, keepdims=True)
            eq3 = t3 <= m3s

            r1 = 1.0 / (jnp.sqrt(m1s) + 1e-8)
            r2 = 1.0 / (jnp.sqrt(m2s) + 1e-8)
            r3 = 1.0 / (jnp.sqrt(m3s) + 1e-8)
            inv_norm = 1.0 / (r1 + r2 + r3)
            wd = (jnp.where(eq1, r1, zero)
                  + jnp.where(eq2, r2, zero)
                  + jnp.where(eq3, r3, zero))

            interp = jax.lax.dot_general(
                wd, points2_ref[0], (((0,), (0,)), ((), ())),
                preferred_element_type=jnp.float32)
            interp = interp * jnp.transpose(inv_norm)
            h = jnp.dot(points1_ref[0, pl.ds(i * _CN, _CN), :],
                        w1t_ref[:_C1, :],
                        preferred_element_type=jnp.float32)
            h = h + jnp.dot(interp, w1t_ref[_C1:, :],
                            preferred_element_type=jnp.float32)
            h = h + b1_ref[...]
            h1_ref[pl.ds(b * _N + i * _CN, _CN), :] = h

            return (jnp.sum(h, axis=0, keepdims=True),
                    jnp.sum(h * h, axis=0, keepdims=True))

        parts = [chunk(i) for i in range(_N // _CN)]
        ps = parts[0][0]
        pq = parts[0][1]
        for cps, cpq in parts[1:]:
            ps = ps + cps
            pq = pq + cpq

        @pl.when(b == 0)
        def _():
            s1_ref[...] = ps
            q1_ref[...] = pq

        @pl.when(b != 0)
        def _():
            s1_ref[...] = s1_ref[...] + ps
            q1_ref[...] = q1_ref[...] + pq

    @pl.when(p == 1)
    def _phase1():
        inv_bn = jnp.float32(1.0 / _BN)
        mean = s1_ref[...] * inv_bn
        var = q1_ref[...] * inv_bn - mean * mean
        scale = g1_ref[...] / jnp.sqrt(var + 1e-5)
        shift = be1_ref[...] - mean * scale
        w2t = w2t_ref[...]
        bias2 = b2_ref[...]

        def chunk(j):
            h = h1_ref[pl.ds(b * _N + j * _CN, _CN), :]
            a = jnp.maximum(h * scale + shift, 0.0)
            h2 = jnp.dot(a, w2t,
                         preferred_element_type=jnp.float32) + bias2
            h2_ref[pl.ds(b * _N + j * _CN, _CN), :] = h2
            return (jnp.sum(h2, axis=0, keepdims=True),
                    jnp.sum(h2 * h2, axis=0, keepdims=True))

        parts = [chunk(j) for j in range(_N // _CN)]
        ps = parts[0][0]
        pq = parts[0][1]
        for cps, cpq in parts[1:]:
            ps = ps + cps
            pq = pq + cpq

        @pl.when(b == 0)
        def _():
            s2_ref[...] = ps
            q2_ref[...] = pq

        @pl.when(b != 0)
        def _():
            s2_ref[...] = s2_ref[...] + ps
            q2_ref[...] = q2_ref[...] + pq

    @pl.when(p == 2)
    def _phase2():
        inv_bn = jnp.float32(1.0 / _BN)
        mean = s2_ref[...] * inv_bn
        var = q2_ref[...] * inv_bn - mean * mean
        scale = g2_ref[...] / jnp.sqrt(var + 1e-5)
        shift = be2_ref[...] - mean * scale

        def chunk(j):
            h = h2_ref[pl.ds(b * _N + j * _CN, _CN), :]
            out_ref[0, pl.ds(j * _CN, _CN), :] = jnp.maximum(
                h * scale + shift, 0.0)

        for j in range(_N // _CN):
            chunk(j)


def kernel(xyz1, xyz2, points1, points2, W1, b1, g1, be1, W2, b2, g2, be2):
    def _p0map(p, b):
        return (jnp.where(p == 0, b, 0), 0, 0)

    def _p2map(p, b):
        return (jnp.where(p == 2, b, 0), 0, 0)

    grid = (3, _B)
    out = pl.pallas_call(
        _fp_kernel,
        grid=grid,
        in_specs=[
            pl.BlockSpec((1, _N, 3), _p0map),
            pl.BlockSpec((1, _S, 3), _p0map),
            pl.BlockSpec((1, _N, _C1), _p0map),
            pl.BlockSpec((1, _S, _C2), _p0map),
            pl.BlockSpec((_H1, _C1 + _C2), lambda p, b: (0, 0)),
            pl.BlockSpec((1, _H1), lambda p, b: (0, 0)),
            pl.BlockSpec((1, _H1), lambda p, b: (0, 0)),
            pl.BlockSpec((1, _H1), lambda p, b: (0, 0)),
            pl.BlockSpec((_H2, _H1), lambda p, b: (0, 0)),
            pl.BlockSpec((1, _H2), lambda p, b: (0, 0)),
            pl.BlockSpec((1, _H2), lambda p, b: (0, 0)),
            pl.BlockSpec((1, _H2), lambda p, b: (0, 0)),
        ],
        out_specs=pl.BlockSpec((1, _N, _H2), _p2map),
        out_shape=jax.ShapeDtypeStruct((_B, _N, _H2), jnp.float32),
        scratch_shapes=[
            pltpu.VMEM((_BN, _H1), jnp.float32),
            pltpu.VMEM((_BN, _H2), jnp.float32),
            pltpu.VMEM((1, _H1), jnp.float32),
            pltpu.VMEM((1, _H1), jnp.float32),
            pltpu.VMEM((1, _H2), jnp.float32),
            pltpu.VMEM((1, _H2), jnp.float32),
            pltpu.VMEM((_C1 + _C2, _H1), jnp.float32),
            pltpu.VMEM((_H1, _H2), jnp.float32),
            pltpu.VMEM((3, _N), jnp.float32),
        ],
    )(xyz1, xyz2, points1, points2,
      W1, b1.reshape(1, _H1), g1.reshape(1, _H1), be1.reshape(1, _H1),
      W2, b2.reshape(1, _H2), g2.reshape(1, _H2), be2.reshape(1, _H2))
    return out


# speedup vs baseline: 1.1352x; 1.1352x over previous
import jax
import jax.numpy as jnp
from jax.experimental import pallas as pl
from jax.experimental.pallas import tpu as pltpu

_B, _N, _S = 8, 2048, 512
_C1, _C2 = 128, 256
_H1, _H2 = 256, 128
_BN = _B * _N
_CN = 256


def _fp_kernel(xyz1_ref, xyz2t_ref, points1_ref, points2_ref,
               w1_ref, b1_ref, g1_ref, be1_ref,
               w2_ref, b2_ref, g2_ref, be2_ref,
               out_ref, h1_ref, h2_ref, s1_ref, q1_ref, s2_ref, q2_ref,
               w1t_ref, w2t_ref):
    p = pl.program_id(0)
    b = pl.program_id(1)

    @pl.when(jnp.logical_and(p == 0, b == 0))
    def _transpose_weights():
        w1t_ref[...] = jnp.transpose(w1_ref[...])
        w2t_ref[...] = jnp.transpose(w2_ref[...])

    @pl.when(p == 0)
    def _phase0():
        x2 = xyz2t_ref[0]
        xc0 = x2[:, 0:1]
        xc1 = x2[:, 1:2]
        xc2 = x2[:, 2:3]
        big = jnp.float32(3.0e38)
        zero = jnp.float32(0.0)

        def chunk(i):
            x1c = xyz1_ref[0, :, pl.ds(i * _CN, _CN)]
            dx = xc0 - x1c[0:1, :]
            dy = xc1 - x1c[1:2, :]
            dz = xc2 - x1c[2:3, :]
            dsq = dx * dx + dy * dy + dz * dz

            m1s = jnp.min(dsq, axis=0, keepdims=True)
            eq1 = dsq <= m1s
            t2 = jnp.where(eq1, big, dsq)
            m2s = jnp.min(t2, axis=0, keepdims=True)
            eq2 = t2 <= m2s
            t3 = jnp.where(eq2, big, t2)
            m3s = jnp.min(t3, axis=0, keepdims=True)
            eq3 = t3 <= m3s

            r1 = 1.0 / (jnp.sqrt(m1s) + 1e-8)
            r2 = 1.0 / (jnp.sqrt(m2s) + 1e-8)
            r3 = 1.0 / (jnp.sqrt(m3s) + 1e-8)
            inv_norm = 1.0 / (r1 + r2 + r3)
            wd = (jnp.where(eq1, r1, zero)
                  + jnp.where(eq2, r2, zero)
                  + jnp.where(eq3, r3, zero))

            interp = jax.lax.dot_general(
                wd, points2_ref[0], (((0,), (0,)), ((), ())),
                preferred_element_type=jnp.float32)
            interp = interp * jnp.transpose(inv_norm)
            h = jnp.dot(points1_ref[0, pl.ds(i * _CN, _CN), :],
                        w1t_ref[:_C1, :],
                        preferred_element_type=jnp.float32)
            h = h + jnp.dot(interp, w1t_ref[_C1:, :],
                            preferred_element_type=jnp.float32)
            h = h + b1_ref[...]
            h1_ref[pl.ds(b * _N + i * _CN, _CN), :] = h

            return (jnp.sum(h, axis=0, keepdims=True),
                    jnp.sum(h * h, axis=0, keepdims=True))

        parts = [chunk(i) for i in range(_N // _CN)]
        ps = parts[0][0]
        pq = parts[0][1]
        for cps, cpq in parts[1:]:
            ps = ps + cps
            pq = pq + cpq

        @pl.when(b == 0)
        def _():
            s1_ref[...] = ps
            q1_ref[...] = pq

        @pl.when(b != 0)
        def _():
            s1_ref[...] = s1_ref[...] + ps
            q1_ref[...] = q1_ref[...] + pq

    @pl.when(p == 1)
    def _phase1():
        inv_bn = jnp.float32(1.0 / _BN)
        mean = s1_ref[...] * inv_bn
        var = q1_ref[...] * inv_bn - mean * mean
        scale = g1_ref[...] / jnp.sqrt(var + 1e-5)
        shift = be1_ref[...] - mean * scale
        w2t = w2t_ref[...]
        bias2 = b2_ref[...]

        def chunk(j):
            h = h1_ref[pl.ds(b * _N + j * _CN, _CN), :]
            a = jnp.maximum(h * scale + shift, 0.0)
            h2 = jnp.dot(a, w2t,
                         preferred_element_type=jnp.float32) + bias2
            h2_ref[pl.ds(b * _N + j * _CN, _CN), :] = h2
            return (jnp.sum(h2, axis=0, keepdims=True),
                    jnp.sum(h2 * h2, axis=0, keepdims=True))

        parts = [chunk(j) for j in range(_N // _CN)]
        ps = parts[0][0]
        pq = parts[0][1]
        for cps, cpq in parts[1:]:
            ps = ps + cps
            pq = pq + cpq

        @pl.when(b == 0)
        def _():
            s2_ref[...] = ps
            q2_ref[...] = pq

        @pl.when(b != 0)
        def _():
            s2_ref[...] = s2_ref[...] + ps
            q2_ref[...] = q2_ref[...] + pq

    @pl.when(p == 2)
    def _phase2():
        inv_bn = jnp.float32(1.0 / _BN)
        mean = s2_ref[...] * inv_bn
        var = q2_ref[...] * inv_bn - mean * mean
        scale = g2_ref[...] / jnp.sqrt(var + 1e-5)
        shift = be2_ref[...] - mean * scale

        def chunk(j):
            h = h2_ref[pl.ds(b * _N + j * _CN, _CN), :]
            out_ref[0, pl.ds(j * _CN, _CN), :] = jnp.maximum(
                h * scale + shift, 0.0)

        for j in range(_N // _CN):
            chunk(j)


def kernel(xyz1, xyz2, points1, points2, W1, b1, g1, be1, W2, b2, g2, be2):
    xyz1t = jnp.transpose(xyz1, (0, 2, 1))

    def _p0map(p, b):
        return (jnp.where(p == 0, b, 0), 0, 0)

    def _p2map(p, b):
        return (jnp.where(p == 2, b, 0), 0, 0)

    grid = (3, _B)
    out = pl.pallas_call(
        _fp_kernel,
        grid=grid,
        in_specs=[
            pl.BlockSpec((1, 3, _N), _p0map),
            pl.BlockSpec((1, _S, 3), _p0map),
            pl.BlockSpec((1, _N, _C1), _p0map),
            pl.BlockSpec((1, _S, _C2), _p0map),
            pl.BlockSpec((_H1, _C1 + _C2), lambda p, b: (0, 0)),
            pl.BlockSpec((1, _H1), lambda p, b: (0, 0)),
            pl.BlockSpec((1, _H1), lambda p, b: (0, 0)),
            pl.BlockSpec((1, _H1), lambda p, b: (0, 0)),
            pl.BlockSpec((_H2, _H1), lambda p, b: (0, 0)),
            pl.BlockSpec((1, _H2), lambda p, b: (0, 0)),
            pl.BlockSpec((1, _H2), lambda p, b: (0, 0)),
            pl.BlockSpec((1, _H2), lambda p, b: (0, 0)),
        ],
        out_specs=pl.BlockSpec((1, _N, _H2), _p2map),
        out_shape=jax.ShapeDtypeStruct((_B, _N, _H2), jnp.float32),
        scratch_shapes=[
            pltpu.VMEM((_BN, _H1), jnp.float32),
            pltpu.VMEM((_BN, _H2), jnp.float32),
            pltpu.VMEM((1, _H1), jnp.float32),
            pltpu.VMEM((1, _H1), jnp.float32),
            pltpu.VMEM((1, _H2), jnp.float32),
            pltpu.VMEM((1, _H2), jnp.float32),
            pltpu.VMEM((_C1 + _C2, _H1), jnp.float32),
            pltpu.VMEM((_H1, _H2), jnp.float32),
        ],
    )(xyz1t, xyz2, points1, points2,
      W1, b1.reshape(1, _H1), g1.reshape(1, _H1), be1.reshape(1, _H1),
      W2, b2.reshape(1, _H2), g2.reshape(1, _H2), be2.reshape(1, _H2))
    return out


# speedup vs baseline: 1.1359x; 1.0006x over previous
import jax
import jax.numpy as jnp
from jax.experimental import pallas as pl
from jax.experimental.pallas import tpu as pltpu

_B, _N, _S = 8, 2048, 512
_C1, _C2 = 128, 256
_H1, _H2 = 256, 128
_BN = _B * _N
_CN = 512


def _fp_kernel(xyz1_ref, xyz2t_ref, points1_ref, points2_ref,
               w1_ref, b1_ref, g1_ref, be1_ref,
               w2_ref, b2_ref, g2_ref, be2_ref,
               out_ref, h1_ref, h2_ref, s1_ref, q1_ref, s2_ref, q2_ref,
               w1t_ref, w2t_ref):
    p = pl.program_id(0)
    b = pl.program_id(1)

    @pl.when(jnp.logical_and(p == 0, b == 0))
    def _transpose_weights():
        w1t_ref[...] = jnp.transpose(w1_ref[...])
        w2t_ref[...] = jnp.transpose(w2_ref[...])

    @pl.when(p == 0)
    def _phase0():
        x2 = xyz2t_ref[0]
        xc0 = x2[:, 0:1]
        xc1 = x2[:, 1:2]
        xc2 = x2[:, 2:3]
        big = jnp.float32(3.0e38)
        zero = jnp.float32(0.0)

        def chunk(i):
            x1c = xyz1_ref[0, :, pl.ds(i * _CN, _CN)]
            dx = xc0 - x1c[0:1, :]
            dy = xc1 - x1c[1:2, :]
            dz = xc2 - x1c[2:3, :]
            dsq = dx * dx + dy * dy + dz * dz

            m1s = jnp.min(dsq, axis=0, keepdims=True)
            eq1 = dsq <= m1s
            t2 = jnp.where(eq1, big, dsq)
            m2s = jnp.min(t2, axis=0, keepdims=True)
            eq2 = t2 <= m2s
            t3 = jnp.where(eq2, big, t2)
            m3s = jnp.min(t3, axis=0, keepdims=True)
            eq3 = t3 <= m3s

            r1 = 1.0 / (jnp.sqrt(m1s) + 1e-8)
            r2 = 1.0 / (jnp.sqrt(m2s) + 1e-8)
            r3 = 1.0 / (jnp.sqrt(m3s) + 1e-8)
            inv_norm = 1.0 / (r1 + r2 + r3)
            wd = (jnp.where(eq1, r1, zero)
                  + jnp.where(eq2, r2, zero)
                  + jnp.where(eq3, r3, zero))

            interp = jax.lax.dot_general(
                wd, points2_ref[0], (((0,), (0,)), ((), ())),
                preferred_element_type=jnp.float32)
            interp = interp * jnp.transpose(inv_norm)
            h = jnp.dot(points1_ref[0, pl.ds(i * _CN, _CN), :],
                        w1t_ref[:_C1, :],
                        preferred_element_type=jnp.float32)
            h = h + jnp.dot(interp, w1t_ref[_C1:, :],
                            preferred_element_type=jnp.float32)
            h = h + b1_ref[...]
            h1_ref[pl.ds(b * _N + i * _CN, _CN), :] = h

            return (jnp.sum(h, axis=0, keepdims=True),
                    jnp.sum(h * h, axis=0, keepdims=True))

        parts = [chunk(i) for i in range(_N // _CN)]
        ps = parts[0][0]
        pq = parts[0][1]
        for cps, cpq in parts[1:]:
            ps = ps + cps
            pq = pq + cpq

        @pl.when(b == 0)
        def _():
            s1_ref[...] = ps
            q1_ref[...] = pq

        @pl.when(b != 0)
        def _():
            s1_ref[...] = s1_ref[...] + ps
            q1_ref[...] = q1_ref[...] + pq

    @pl.when(p == 1)
    def _phase1():
        inv_bn = jnp.float32(1.0 / _BN)
        mean = s1_ref[...] * inv_bn
        var = q1_ref[...] * inv_bn - mean * mean
        scale = g1_ref[...] / jnp.sqrt(var + 1e-5)
        shift = be1_ref[...] - mean * scale
        w2t = w2t_ref[...]
        bias2 = b2_ref[...]

        def chunk(j):
            h = h1_ref[pl.ds(b * _N + j * _CN, _CN), :]
            a = jnp.maximum(h * scale + shift, 0.0)
            h2 = jnp.dot(a, w2t,
                         preferred_element_type=jnp.float32) + bias2
            h2_ref[pl.ds(b * _N + j * _CN, _CN), :] = h2
            return (jnp.sum(h2, axis=0, keepdims=True),
                    jnp.sum(h2 * h2, axis=0, keepdims=True))

        parts = [chunk(j) for j in range(_N // _CN)]
        ps = parts[0][0]
        pq = parts[0][1]
        for cps, cpq in parts[1:]:
            ps = ps + cps
            pq = pq + cpq

        @pl.when(b == 0)
        def _():
            s2_ref[...] = ps
            q2_ref[...] = pq

        @pl.when(b != 0)
        def _():
            s2_ref[...] = s2_ref[...] + ps
            q2_ref[...] = q2_ref[...] + pq

    @pl.when(p == 2)
    def _phase2():
        inv_bn = jnp.float32(1.0 / _BN)
        mean = s2_ref[...] * inv_bn
        var = q2_ref[...] * inv_bn - mean * mean
        scale = g2_ref[...] / jnp.sqrt(var + 1e-5)
        shift = be2_ref[...] - mean * scale

        def chunk(j):
            h = h2_ref[pl.ds(b * _N + j * _CN, _CN), :]
            out_ref[0, pl.ds(j * _CN, _CN), :] = jnp.maximum(
                h * scale + shift, 0.0)

        for j in range(_N // _CN):
            chunk(j)


def kernel(xyz1, xyz2, points1, points2, W1, b1, g1, be1, W2, b2, g2, be2):
    xyz1t = jnp.transpose(xyz1, (0, 2, 1))

    def _p0map(p, b):
        return (jnp.where(p == 0, b, 0), 0, 0)

    def _p2map(p, b):
        return (jnp.where(p == 2, b, 0), 0, 0)

    grid = (3, _B)
    out = pl.pallas_call(
        _fp_kernel,
        grid=grid,
        in_specs=[
            pl.BlockSpec((1, 3, _N), _p0map),
            pl.BlockSpec((1, _S, 3), _p0map),
            pl.BlockSpec((1, _N, _C1), _p0map),
            pl.BlockSpec((1, _S, _C2), _p0map),
            pl.BlockSpec((_H1, _C1 + _C2), lambda p, b: (0, 0)),
            pl.BlockSpec((1, _H1), lambda p, b: (0, 0)),
            pl.BlockSpec((1, _H1), lambda p, b: (0, 0)),
            pl.BlockSpec((1, _H1), lambda p, b: (0, 0)),
            pl.BlockSpec((_H2, _H1), lambda p, b: (0, 0)),
            pl.BlockSpec((1, _H2), lambda p, b: (0, 0)),
            pl.BlockSpec((1, _H2), lambda p, b: (0, 0)),
            pl.BlockSpec((1, _H2), lambda p, b: (0, 0)),
        ],
        out_specs=pl.BlockSpec((1, _N, _H2), _p2map),
        out_shape=jax.ShapeDtypeStruct((_B, _N, _H2), jnp.float32),
        scratch_shapes=[
            pltpu.VMEM((_BN, _H1), jnp.float32),
            pltpu.VMEM((_BN, _H2), jnp.float32),
            pltpu.VMEM((1, _H1), jnp.float32),
            pltpu.VMEM((1, _H1), jnp.float32),
            pltpu.VMEM((1, _H2), jnp.float32),
            pltpu.VMEM((1, _H2), jnp.float32),
            pltpu.VMEM((_C1 + _C2, _H1), jnp.float32),
            pltpu.VMEM((_H1, _H2), jnp.float32),
        ],
    )(xyz1t, xyz2, points1, points2,
      W1, b1.reshape(1, _H1), g1.reshape(1, _H1), be1.reshape(1, _H1),
      W2, b2.reshape(1, _H2), g2.reshape(1, _H2), be2.reshape(1, _H2))
    return out
